# baseline (speedup 1.0000x reference)
import jax
import jax.numpy as jnp
from jax import lax
from jax.experimental import pallas as pl
from jax.experimental.pallas import tpu as pltpu
from jax.experimental.pallas import tpu_sc as plsc

N = 10000
E = 160000
HID = 512
CW = HID // 2
NC = 2
NS = 16
LB = 128
NBATCH = 1280
EP = NBATCH * LB
NBT = NBATCH // NS
HB = NBT // 2
RPT = N // NS
NPAD = N + 8

_DT = jnp.bfloat16


def _make_segsum():
    def body(t0, t1, srcb, dstb, zeros, o0, o1,
             idx_s, idx_d, rows0, rows1, acc, sem0, sem1):
        tables = (t0, t1)
        outs = (o0, o1)
        c = lax.axis_index("c")
        s = lax.axis_index("s")
        my_rows = pl.ds(s * RPT, RPT)

        for cc in range(NC):
            @pl.when(c == cc)
            def _(cc=cc):
                tab = tables[cc]
                out = outs[cc]
                pltpu.sync_copy(zeros.at[my_rows], acc.at[my_rows])
                plsc.subcore_barrier()

                for hh in range(2):
                    pltpu.sync_copy(
                        srcb.at[pl.ds((s * 2 + hh) * HB, HB)], idx_s)
                    pltpu.sync_copy(
                        dstb.at[pl.ds((s * 2 + hh) * HB, HB)], idx_d)
                    pltpu.async_copy(tab.at[idx_s.at[0]], rows0, sem0)
                    pltpu.async_copy(tab.at[idx_s.at[1]], rows1, sem1)

                    def ebody(i, carry, tab=tab):
                        b = 2 * i
                        pltpu.make_async_copy(tab.at[idx_s.at[b]],
                                              rows0, sem0).wait()
                        pltpu.sync_copy(rows0, acc.at[idx_d.at[b]], add=True)
                        pltpu.async_copy(tab.at[idx_s.at[b + 2]],
                                         rows0, sem0)
                        pltpu.make_async_copy(tab.at[idx_s.at[b + 1]],
                                              rows1, sem1).wait()
                        pltpu.sync_copy(rows1, acc.at[idx_d.at[b + 1]],
                                        add=True)
                        pltpu.async_copy(tab.at[idx_s.at[b + 3]],
                                         rows1, sem1)
                        return carry

                    lax.fori_loop(0, HB // 2 - 1, ebody, 0)
                    b = HB - 2
                    pltpu.make_async_copy(tab.at[idx_s.at[b]],
                                          rows0, sem0).wait()
                    pltpu.sync_copy(rows0, acc.at[idx_d.at[b]], add=True)
                    pltpu.make_async_copy(tab.at[idx_s.at[b + 1]],
                                          rows1, sem1).wait()
                    pltpu.sync_copy(rows1, acc.at[idx_d.at[b + 1]], add=True)

                plsc.subcore_barrier()
                pltpu.sync_copy(acc.at[my_rows], out.at[my_rows])

    mesh = plsc.VectorSubcoreMesh(
        core_axis_name="c", subcore_axis_name="s",
        num_cores=NC, num_subcores=NS)
    return pl.kernel(
        body,
        out_type=[jax.ShapeDtypeStruct((N, CW), _DT)] * 2,
        mesh=mesh,
        compiler_params=pltpu.CompilerParams(use_tc_tiling_on_sc=False),
        scratch_types=[
            pltpu.VMEM((HB, LB), jnp.int32),
            pltpu.VMEM((HB, LB), jnp.int32),
            pltpu.VMEM((LB, CW), _DT),
            pltpu.VMEM((LB, CW), _DT),
            pltpu.VMEM_SHARED((NPAD, CW), _DT),
            pltpu.SemaphoreType.DMA,
            pltpu.SemaphoreType.DMA,
        ],
    )


_BN = 2000
_NB = N // _BN


def _make_tc_z(din):
    def body(x, wn, z0, z1):
        z = jnp.dot(x[...], wn[...],
                    preferred_element_type=jnp.float32).astype(_DT)
        z0[...] = z[:, :CW]
        z1[...] = z[:, CW:]

    return pl.pallas_call(
        body,
        grid=(_NB,),
        in_specs=[pl.BlockSpec((_BN, din), lambda i: (i, 0)),
                  pl.BlockSpec((din, HID), lambda i: (0, 0))],
        out_specs=[pl.BlockSpec((_BN, CW), lambda i: (i, 0))] * 2,
        out_shape=[jax.ShapeDtypeStruct((N, CW), _DT)] * 2,
    )


def _make_tc_combine(din):
    def body(h, s0, s1, ws, b, wnn, ho, z0, z1):
        o = jnp.dot(h[...], ws[...], preferred_element_type=jnp.float32)
        o += jnp.concatenate([s0[...], s1[...]], axis=1).astype(jnp.float32)
        hp = jnp.maximum(o + b[...], 0.0).astype(_DT)
        ho[...] = hp
        z = jnp.dot(hp, wnn[...],
                    preferred_element_type=jnp.float32).astype(_DT)
        z0[...] = z[:, :CW]
        z1[...] = z[:, CW:]

    return pl.pallas_call(
        body,
        grid=(_NB,),
        in_specs=[pl.BlockSpec((_BN, din), lambda i: (i, 0)),
                  pl.BlockSpec((_BN, CW), lambda i: (i, 0)),
                  pl.BlockSpec((_BN, CW), lambda i: (i, 0)),
                  pl.BlockSpec((din, HID), lambda i: (0, 0)),
                  pl.BlockSpec((1, HID), lambda i: (0, 0)),
                  pl.BlockSpec((HID, HID), lambda i: (0, 0))],
        out_specs=[pl.BlockSpec((_BN, HID), lambda i: (i, 0)),
                   pl.BlockSpec((_BN, CW), lambda i: (i, 0)),
                   pl.BlockSpec((_BN, CW), lambda i: (i, 0))],
        out_shape=[jax.ShapeDtypeStruct((N, HID), _DT),
                   jax.ShapeDtypeStruct((N, CW), _DT),
                   jax.ShapeDtypeStruct((N, CW), _DT)],
    )


def _make_tc_final(din):
    def body(h, s0, s1, ws, b, wc, bc, out, acc):
        i = pl.program_id(0)
        o = jnp.dot(h[...], ws[...], preferred_element_type=jnp.float32)
        o += jnp.concatenate([s0[...], s1[...]], axis=1).astype(jnp.float32)
        o = jnp.maximum(o + b[...], 0.0)

        @pl.when(i == 0)
        def _():
            acc[...] = jnp.zeros_like(acc)

        acc[...] += jnp.sum(o, axis=0, keepdims=True)

        @pl.when(i == _NB - 1)
        def _():
            g = acc[...]
            logits = jnp.dot(g, wc[...], preferred_element_type=jnp.float32)
            logits += bc[...]
            m = jnp.max(logits, axis=1, keepdims=True)
            e = jnp.exp(logits - m)
            out[...] = e / jnp.sum(e, axis=1, keepdims=True)

    return pl.pallas_call(
        body,
        grid=(_NB,),
        in_specs=[pl.BlockSpec((_BN, din), lambda i: (i, 0)),
                  pl.BlockSpec((_BN, CW), lambda i: (i, 0)),
                  pl.BlockSpec((_BN, CW), lambda i: (i, 0)),
                  pl.BlockSpec((din, HID), lambda i: (0, 0)),
                  pl.BlockSpec((1, HID), lambda i: (0, 0)),
                  pl.BlockSpec((HID, 32), lambda i: (0, 0)),
                  pl.BlockSpec((1, 32), lambda i: (0, 0))],
        out_specs=pl.BlockSpec((1, 32), lambda i: (0, 0)),
        out_shape=jax.ShapeDtypeStruct((1, 32), jnp.float32),
        scratch_shapes=[pltpu.VMEM((1, 512), jnp.float32)],
    )


def kernel(x, edge_index, Ws0, Wn0, b0, Ws1, Wn1, b1, Ws2, Wn2, b2, Wc, bc):
    pad = EP - E
    srcb = jnp.concatenate(
        [edge_index[0], jnp.zeros((pad,), jnp.int32)]).reshape(NBATCH, LB)
    dstb = jnp.concatenate(
        [edge_index[1], jnp.full((pad,), N, jnp.int32)]).reshape(NBATCH, LB)
    z = jnp.zeros((N, CW), _DT)

    xb = x.astype(_DT)
    segsum = _make_segsum()

    z0 = _make_tc_z(256)(xb, Wn0.astype(_DT))
    s0 = segsum(*z0, srcb, dstb, z)
    h1, *z1 = _make_tc_combine(256)(xb, *s0, Ws0.astype(_DT),
                                    b0.reshape(1, HID), Wn1.astype(_DT))
    s1 = segsum(*z1, srcb, dstb, z)
    h2, *z2 = _make_tc_combine(HID)(h1, *s1, Ws1.astype(_DT),
                                    b1.reshape(1, HID), Wn2.astype(_DT))
    s2 = segsum(*z2, srcb, dstb, z)
    probs = _make_tc_final(HID)(h2, *s2, Ws2.astype(_DT),
                                b2.reshape(1, HID), Wc, bc.reshape(1, 32))
    return probs.reshape(32)


# speedup vs baseline: 1.5914x; 1.5914x over previous
import jax
import jax.numpy as jnp
from jax import lax
from jax.experimental import pallas as pl
from jax.experimental.pallas import tpu as pltpu
from jax.experimental.pallas import tpu_sc as plsc

N = 10000
E = 160000
HID = 512
CW = HID // 2
NC = 2
NS = 16
LB = 112
NBATCH = 1440
EP = NBATCH * LB
NBT = NBATCH // NS
RPT = N // NS
NPAD = N + 8

_DT = jnp.bfloat16


def _make_segsum():
    def body(t0, t1, srcb, dstb, zeros, o0, o1,
             idx_s, idx_d, rows0, rows1, acc, sem0, sem1):
        tables = (t0, t1)
        outs = (o0, o1)
        c = lax.axis_index("c")
        s = lax.axis_index("s")
        my_rows = pl.ds(s * RPT, RPT)

        pltpu.sync_copy(srcb.at[pl.ds(s * NBT, NBT)], idx_s)
        pltpu.sync_copy(dstb.at[pl.ds(s * NBT, NBT)], idx_d)

        for cc in range(NC):
            @pl.when(c == cc)
            def _(cc=cc):
                tab = tables[cc]
                out = outs[cc]
                pltpu.sync_copy(zeros.at[my_rows], acc.at[my_rows])
                plsc.subcore_barrier()

                pltpu.async_copy(tab.at[idx_s.at[0]], rows0, sem0)
                pltpu.async_copy(tab.at[idx_s.at[1]], rows1, sem1)

                def ebody(i, carry, tab=tab):
                    b = 2 * i
                    pltpu.make_async_copy(tab.at[idx_s.at[b]],
                                          rows0, sem0).wait()
                    pltpu.sync_copy(rows0, acc.at[idx_d.at[b]], add=True)
                    pltpu.async_copy(tab.at[idx_s.at[b + 2]], rows0, sem0)
                    pltpu.make_async_copy(tab.at[idx_s.at[b + 1]],
                                          rows1, sem1).wait()
                    pltpu.sync_copy(rows1, acc.at[idx_d.at[b + 1]], add=True)
                    pltpu.async_copy(tab.at[idx_s.at[b + 3]], rows1, sem1)
                    return carry

                lax.fori_loop(0, NBT // 2 - 1, ebody, 0)
                b = NBT - 2
                pltpu.make_async_copy(tab.at[idx_s.at[b]], rows0, sem0).wait()
                pltpu.sync_copy(rows0, acc.at[idx_d.at[b]], add=True)
                pltpu.make_async_copy(tab.at[idx_s.at[b + 1]],
                                      rows1, sem1).wait()
                pltpu.sync_copy(rows1, acc.at[idx_d.at[b + 1]], add=True)

                plsc.subcore_barrier()
                pltpu.sync_copy(acc.at[my_rows], out.at[my_rows])

    mesh = plsc.VectorSubcoreMesh(
        core_axis_name="c", subcore_axis_name="s",
        num_cores=NC, num_subcores=NS)
    return pl.kernel(
        body,
        out_type=[jax.ShapeDtypeStruct((N, CW), _DT)] * 2,
        mesh=mesh,
        compiler_params=pltpu.CompilerParams(use_tc_tiling_on_sc=False),
        scratch_types=[
            pltpu.VMEM((NBT, LB), jnp.int32),
            pltpu.VMEM((NBT, LB), jnp.int32),
            pltpu.VMEM((LB, CW), _DT),
            pltpu.VMEM((LB, CW), _DT),
            pltpu.VMEM_SHARED((NPAD, CW), _DT),
            pltpu.SemaphoreType.DMA,
            pltpu.SemaphoreType.DMA,
        ],
    )


_BN = 2000
_NB = N // _BN


def _make_tc_z(din):
    def body(x, wn, z0, z1):
        z = jnp.dot(x[...], wn[...],
                    preferred_element_type=jnp.float32).astype(_DT)
        z0[...] = z[:, :CW]
        z1[...] = z[:, CW:]

    return pl.pallas_call(
        body,
        grid=(_NB,),
        in_specs=[pl.BlockSpec((_BN, din), lambda i: (i, 0)),
                  pl.BlockSpec((din, HID), lambda i: (0, 0))],
        out_specs=[pl.BlockSpec((_BN, CW), lambda i: (i, 0))] * 2,
        out_shape=[jax.ShapeDtypeStruct((N, CW), _DT)] * 2,
    )


def _make_tc_self(din):
    def body(h, ws, b, u):
        o = jnp.dot(h[...], ws[...], preferred_element_type=jnp.float32)
        u[...] = (o + b[...]).astype(_DT)

    return pl.pallas_call(
        body,
        grid=(_NB,),
        in_specs=[pl.BlockSpec((_BN, din), lambda i: (i, 0)),
                  pl.BlockSpec((din, HID), lambda i: (0, 0)),
                  pl.BlockSpec((1, HID), lambda i: (0, 0))],
        out_specs=pl.BlockSpec((_BN, HID), lambda i: (i, 0)),
        out_shape=jax.ShapeDtypeStruct((N, HID), _DT),
    )


def _make_tc_post():
    def body(u, s0, s1, wnn, ho, z0, z1):
        o = u[...].astype(jnp.float32)
        o += jnp.concatenate([s0[...], s1[...]], axis=1).astype(jnp.float32)
        hp = jnp.maximum(o, 0.0).astype(_DT)
        ho[...] = hp
        z = jnp.dot(hp, wnn[...],
                    preferred_element_type=jnp.float32).astype(_DT)
        z0[...] = z[:, :CW]
        z1[...] = z[:, CW:]

    return pl.pallas_call(
        body,
        grid=(_NB,),
        in_specs=[pl.BlockSpec((_BN, HID), lambda i: (i, 0)),
                  pl.BlockSpec((_BN, CW), lambda i: (i, 0)),
                  pl.BlockSpec((_BN, CW), lambda i: (i, 0)),
                  pl.BlockSpec((HID, HID), lambda i: (0, 0))],
        out_specs=[pl.BlockSpec((_BN, HID), lambda i: (i, 0)),
                   pl.BlockSpec((_BN, CW), lambda i: (i, 0)),
                   pl.BlockSpec((_BN, CW), lambda i: (i, 0))],
        out_shape=[jax.ShapeDtypeStruct((N, HID), _DT),
                   jax.ShapeDtypeStruct((N, CW), _DT),
                   jax.ShapeDtypeStruct((N, CW), _DT)],
    )


def _make_tc_final():
    def body(u, s0, s1, wc, bc, out, acc):
        i = pl.program_id(0)
        o = u[...].astype(jnp.float32)
        o += jnp.concatenate([s0[...], s1[...]], axis=1).astype(jnp.float32)
        o = jnp.maximum(o, 0.0)

        @pl.when(i == 0)
        def _():
            acc[...] = jnp.zeros_like(acc)

        acc[...] += jnp.sum(o, axis=0, keepdims=True)

        @pl.when(i == _NB - 1)
        def _():
            g = acc[...]
            logits = jnp.dot(g, wc[...], preferred_element_type=jnp.float32)
            logits += bc[...]
            m = jnp.max(logits, axis=1, keepdims=True)
            e = jnp.exp(logits - m)
            out[...] = e / jnp.sum(e, axis=1, keepdims=True)

    return pl.pallas_call(
        body,
        grid=(_NB,),
        in_specs=[pl.BlockSpec((_BN, HID), lambda i: (i, 0)),
                  pl.BlockSpec((_BN, CW), lambda i: (i, 0)),
                  pl.BlockSpec((_BN, CW), lambda i: (i, 0)),
                  pl.BlockSpec((HID, 32), lambda i: (0, 0)),
                  pl.BlockSpec((1, 32), lambda i: (0, 0))],
        out_specs=pl.BlockSpec((1, 32), lambda i: (0, 0)),
        out_shape=jax.ShapeDtypeStruct((1, 32), jnp.float32),
        scratch_shapes=[pltpu.VMEM((1, 512), jnp.float32)],
    )


def kernel(x, edge_index, Ws0, Wn0, b0, Ws1, Wn1, b1, Ws2, Wn2, b2, Wc, bc):
    pad = EP - E
    srcb = jnp.concatenate(
        [edge_index[0], jnp.zeros((pad,), jnp.int32)]).reshape(NBATCH, LB)
    dstb = jnp.concatenate(
        [edge_index[1], jnp.full((pad,), N, jnp.int32)]).reshape(NBATCH, LB)
    z = jnp.zeros((N, CW), _DT)

    xb = x.astype(_DT)
    segsum = _make_segsum()
    tc_self = _make_tc_self(HID)
    tc_post = _make_tc_post()

    z0 = _make_tc_z(256)(xb, Wn0.astype(_DT))
    u0 = _make_tc_self(256)(xb, Ws0.astype(_DT), b0.reshape(1, HID))
    s0 = segsum(*z0, srcb, dstb, z)
    h1, *z1 = tc_post(u0, *s0, Wn1.astype(_DT))
    u1 = tc_self(h1, Ws1.astype(_DT), b1.reshape(1, HID))
    s1 = segsum(*z1, srcb, dstb, z)
    h2, *z2 = tc_post(u1, *s1, Wn2.astype(_DT))
    u2 = tc_self(h2, Ws2.astype(_DT), b2.reshape(1, HID))
    s2 = segsum(*z2, srcb, dstb, z)
    probs = _make_tc_final()(u2, *s2, Wc, bc.reshape(1, 32))
    return probs.reshape(32)


# speedup vs baseline: 1.9532x; 1.2273x over previous
import jax
import jax.numpy as jnp
from jax import lax
from jax.experimental import pallas as pl
from jax.experimental.pallas import tpu as pltpu
from jax.experimental.pallas import tpu_sc as plsc

N = 10000
E = 160000
HID = 512
CW = HID // 2
NC = 2
NS = 16
LB = 80
NBATCH = E // LB
NBT = NBATCH // NS
RPT = N // NS
NPAD = N

_DT = jnp.bfloat16


def _make_segsum():
    def body(t0, t1, srcb, dstb, zeros, o0, o1,
             idx_s, idx_d, rows0, rows1, acc, sem0, sem1):
        tables = (t0, t1)
        outs = (o0, o1)
        c = lax.axis_index("c")
        s = lax.axis_index("s")
        my_rows = pl.ds(s * RPT, RPT)

        pltpu.sync_copy(srcb.at[pl.ds(s * NBT, NBT)], idx_s)
        pltpu.sync_copy(dstb.at[pl.ds(s * NBT, NBT)], idx_d)

        for cc in range(NC):
            @pl.when(c == cc)
            def _(cc=cc):
                tab = tables[cc]
                out = outs[cc]
                pltpu.sync_copy(zeros.at[my_rows], acc.at[my_rows])
                plsc.subcore_barrier()

                if NBT % 2:
                    pltpu.async_copy(tab.at[idx_s.at[NBT - 1]], rows0, sem0)
                    pltpu.make_async_copy(tab.at[idx_s.at[NBT - 1]],
                                          rows0, sem0).wait()
                    pltpu.sync_copy(rows0, acc.at[idx_d.at[NBT - 1]],
                                    add=True)

                pltpu.async_copy(tab.at[idx_s.at[0]], rows0, sem0)
                pltpu.async_copy(tab.at[idx_s.at[1]], rows1, sem1)

                def ebody(i, carry, tab=tab):
                    b = 2 * i
                    pltpu.make_async_copy(tab.at[idx_s.at[b]],
                                          rows0, sem0).wait()
                    pltpu.sync_copy(rows0, acc.at[idx_d.at[b]], add=True)
                    pltpu.async_copy(tab.at[idx_s.at[b + 2]], rows0, sem0)
                    pltpu.make_async_copy(tab.at[idx_s.at[b + 1]],
                                          rows1, sem1).wait()
                    pltpu.sync_copy(rows1, acc.at[idx_d.at[b + 1]], add=True)
                    pltpu.async_copy(tab.at[idx_s.at[b + 3]], rows1, sem1)
                    return carry

                lax.fori_loop(0, (NBT - NBT % 2) // 2 - 1, ebody, 0)
                b = NBT - NBT % 2 - 2
                pltpu.make_async_copy(tab.at[idx_s.at[b]], rows0, sem0).wait()
                pltpu.sync_copy(rows0, acc.at[idx_d.at[b]], add=True)
                pltpu.make_async_copy(tab.at[idx_s.at[b + 1]],
                                      rows1, sem1).wait()
                pltpu.sync_copy(rows1, acc.at[idx_d.at[b + 1]], add=True)

                plsc.subcore_barrier()
                pltpu.sync_copy(acc.at[my_rows], out.at[my_rows])

    mesh = plsc.VectorSubcoreMesh(
        core_axis_name="c", subcore_axis_name="s",
        num_cores=NC, num_subcores=NS)
    return pl.kernel(
        body,
        out_type=[jax.ShapeDtypeStruct((N, CW), _DT)] * 2,
        mesh=mesh,
        compiler_params=pltpu.CompilerParams(use_tc_tiling_on_sc=False),
        scratch_types=[
            pltpu.VMEM((NBT, LB), jnp.int32),
            pltpu.VMEM((NBT, LB), jnp.int32),
            pltpu.VMEM((LB, CW), _DT),
            pltpu.VMEM((LB, CW), _DT),
            pltpu.VMEM_SHARED((NPAD, CW), _DT),
            pltpu.SemaphoreType.DMA,
            pltpu.SemaphoreType.DMA,
        ],
    )


_BN = 2000
_NB = N // _BN


def _make_tc_z(din):
    def body(x, wn, z0, z1):
        z = jnp.dot(x[...], wn[...],
                    preferred_element_type=jnp.float32).astype(_DT)
        z0[...] = z[:, :CW]
        z1[...] = z[:, CW:]

    return pl.pallas_call(
        body,
        grid=(_NB,),
        in_specs=[pl.BlockSpec((_BN, din), lambda i: (i, 0)),
                  pl.BlockSpec((din, HID), lambda i: (0, 0))],
        out_specs=[pl.BlockSpec((_BN, CW), lambda i: (i, 0))] * 2,
        out_shape=[jax.ShapeDtypeStruct((N, CW), _DT)] * 2,
    )


def _make_tc_self(din):
    def body(h, ws, b, u):
        o = jnp.dot(h[...], ws[...], preferred_element_type=jnp.float32)
        u[...] = (o + b[...]).astype(_DT)

    return pl.pallas_call(
        body,
        grid=(_NB,),
        in_specs=[pl.BlockSpec((_BN, din), lambda i: (i, 0)),
                  pl.BlockSpec((din, HID), lambda i: (0, 0)),
                  pl.BlockSpec((1, HID), lambda i: (0, 0))],
        out_specs=pl.BlockSpec((_BN, HID), lambda i: (i, 0)),
        out_shape=jax.ShapeDtypeStruct((N, HID), _DT),
    )


def _make_tc_post():
    def body(u, s0, s1, wnn, ho, z0, z1):
        o = u[...].astype(jnp.float32)
        o += jnp.concatenate([s0[...], s1[...]], axis=1).astype(jnp.float32)
        hp = jnp.maximum(o, 0.0).astype(_DT)
        ho[...] = hp
        z = jnp.dot(hp, wnn[...],
                    preferred_element_type=jnp.float32).astype(_DT)
        z0[...] = z[:, :CW]
        z1[...] = z[:, CW:]

    return pl.pallas_call(
        body,
        grid=(_NB,),
        in_specs=[pl.BlockSpec((_BN, HID), lambda i: (i, 0)),
                  pl.BlockSpec((_BN, CW), lambda i: (i, 0)),
                  pl.BlockSpec((_BN, CW), lambda i: (i, 0)),
                  pl.BlockSpec((HID, HID), lambda i: (0, 0))],
        out_specs=[pl.BlockSpec((_BN, HID), lambda i: (i, 0)),
                   pl.BlockSpec((_BN, CW), lambda i: (i, 0)),
                   pl.BlockSpec((_BN, CW), lambda i: (i, 0))],
        out_shape=[jax.ShapeDtypeStruct((N, HID), _DT),
                   jax.ShapeDtypeStruct((N, CW), _DT),
                   jax.ShapeDtypeStruct((N, CW), _DT)],
    )


def _make_tc_final():
    def body(u, s0, s1, wc, bc, out, acc):
        i = pl.program_id(0)
        o = u[...].astype(jnp.float32)
        o += jnp.concatenate([s0[...], s1[...]], axis=1).astype(jnp.float32)
        o = jnp.maximum(o, 0.0)

        @pl.when(i == 0)
        def _():
            acc[...] = jnp.zeros_like(acc)

        acc[...] += jnp.sum(o, axis=0, keepdims=True)

        @pl.when(i == _NB - 1)
        def _():
            g = acc[...]
            logits = jnp.dot(g, wc[...], preferred_element_type=jnp.float32)
            logits += bc[...]
            m = jnp.max(logits, axis=1, keepdims=True)
            e = jnp.exp(logits - m)
            out[...] = e / jnp.sum(e, axis=1, keepdims=True)

    return pl.pallas_call(
        body,
        grid=(_NB,),
        in_specs=[pl.BlockSpec((_BN, HID), lambda i: (i, 0)),
                  pl.BlockSpec((_BN, CW), lambda i: (i, 0)),
                  pl.BlockSpec((_BN, CW), lambda i: (i, 0)),
                  pl.BlockSpec((HID, 32), lambda i: (0, 0)),
                  pl.BlockSpec((1, 32), lambda i: (0, 0))],
        out_specs=pl.BlockSpec((1, 32), lambda i: (0, 0)),
        out_shape=jax.ShapeDtypeStruct((1, 32), jnp.float32),
        scratch_shapes=[pltpu.VMEM((1, 512), jnp.float32)],
    )


def kernel(x, edge_index, Ws0, Wn0, b0, Ws1, Wn1, b1, Ws2, Wn2, b2, Wc, bc):
    srcb = edge_index[0].reshape(NBATCH, LB)
    dstb = edge_index[1].reshape(NBATCH, LB)
    z = jnp.zeros((N, CW), _DT)

    xb = x.astype(_DT)
    segsum = _make_segsum()
    tc_self = _make_tc_self(HID)
    tc_post = _make_tc_post()

    z0 = _make_tc_z(256)(xb, Wn0.astype(_DT))
    u0 = _make_tc_self(256)(xb, Ws0.astype(_DT), b0.reshape(1, HID))
    s0 = segsum(*z0, srcb, dstb, z)
    h1, *z1 = tc_post(u0, *s0, Wn1.astype(_DT))
    u1 = tc_self(h1, Ws1.astype(_DT), b1.reshape(1, HID))
    s1 = segsum(*z1, srcb, dstb, z)
    h2, *z2 = tc_post(u1, *s1, Wn2.astype(_DT))
    u2 = tc_self(h2, Ws2.astype(_DT), b2.reshape(1, HID))
    s2 = segsum(*z2, srcb, dstb, z)
    probs = _make_tc_final()(u2, *s2, Wc, bc.reshape(1, 32))
    return probs.reshape(32)
